# baseline (speedup 1.0000x reference)
import functools

import jax
import jax.numpy as jnp
from jax import lax
from jax.experimental import pallas as pl
from jax.experimental.pallas import tpu as pltpu
from jax.experimental.pallas import tpu_sc as plsc

_N_EDGES = 320000
_N_RELS = 1000
_HID = 128
_D2 = _HID // 2
_NW = 32
_EPW = _N_EDGES // _NW
_C = 40
_NCHUNK = _EPW // _C
_DEPTH = 4


def _edge_score_body(node_hbm, relp_hbm, src_hbm, dst_hbm, ridx_hbm, out_hbm,
                     rel_res, srcs, dsts, rids, outs, heads, tails,
                     sem_rows, sem_idx, sem_out):
    cid = lax.axis_index("c")
    sid = lax.axis_index("s")
    wid = sid * 2 + cid
    base = wid * _EPW

    pltpu.sync_copy(relp_hbm, rel_res)

    def start_idx(c, r):
        sl = pl.ds(base + c * _C, _C)
        pltpu.async_copy(src_hbm.at[sl], srcs[r], sem_idx[r])
        pltpu.async_copy(dst_hbm.at[sl], dsts[r], sem_idx[r])
        pltpu.async_copy(ridx_hbm.at[sl], rids[r].at[pl.ds(0, _C)],
                         sem_idx[r])

    def wait_idx(c, r):
        sl = pl.ds(base + c * _C, _C)
        pltpu.make_async_copy(src_hbm.at[sl], srcs[r], sem_idx[r]).wait()
        pltpu.make_async_copy(dst_hbm.at[sl], dsts[r], sem_idx[r]).wait()
        pltpu.make_async_copy(ridx_hbm.at[sl], rids[r].at[pl.ds(0, _C)],
                              sem_idx[r]).wait()

    def start_rows(r):
        pltpu.async_copy(node_hbm.at[srcs[r]], heads[r], sem_rows[r])
        pltpu.async_copy(node_hbm.at[dsts[r]], tails[r], sem_rows[r])

    def wait_rows(r):
        pltpu.make_async_copy(node_hbm.at[srcs[r]], heads[r],
                              sem_rows[r]).wait()
        pltpu.make_async_copy(node_hbm.at[dsts[r]], tails[r],
                              sem_rows[r]).wait()

    lane = lax.iota(jnp.int32, 16)
    low4 = lane < 4

    def compute(r):
        head_v, tail_v, rid_v, out_v = heads[r], tails[r], rids[r], outs[r]

        def block_body(b, carry2):
            ids = rid_v[pl.ds(b * 4, 16)]
            vb = jnp.zeros((16,), jnp.float32)
            for u in range(4):
                k = b * 4 + u
                rid = jnp.take_along_axis(
                    ids, jnp.full((16,), u, jnp.int32), axis=0)
                rbase = rid * (_HID // 2) + lane
                bc = lambda v: plsc.bitcast(v, jnp.bfloat16)
                rp = [bc(plsc.load_gather(rel_res, [rbase + q * 16]))
                      for q in range(4)]
                hp = [bc(head_v[k, pl.ds(q * 16, 16)]) for q in range(4)]
                tp = [bc(tail_v[k, pl.ds(q * 16, 16)]) for q in range(4)]
                terms = []
                for q in range(2):
                    fwd = (hp[q] * rp[q]) * tp[q + 2]
                    bwd = (tp[q] * rp[q + 2]) * hp[q + 2]
                    terms.extend(plsc.unpack(
                        fwd, format=plsc.PackFormat.INTERLEAVED))
                    terms.extend(plsc.unpack(
                        bwd, format=plsc.PackFormat.INTERLEAVED))
                while len(terms) > 1:
                    terms = [a + b2 for a, b2 in
                             zip(terms[::2], terms[1::2])]
                s = jnp.full((16,), jnp.sum(terms[0]))
                vb = jnp.where(lane == u, s, vb)
            score = jnp.clip(0.5 * vb, -20.0, 20.0)
            plsc.store_scatter(out_v, [b * 4 + lane], score, mask=low4)
            return carry2

        lax.fori_loop(0, _C // 4, block_body, 0)

    _LOOK = _DEPTH - 1
    for r0 in range(_DEPTH):
        start_idx(r0, r0)
    for r0 in range(_LOOK):
        wait_idx(r0, r0)
        start_rows(r0)

    def ring_body(i, carry):
        for r in range(_DEPTH):
            c = _DEPTH * i + r
            sl = pl.ds(base + c * _C, _C)

            @pl.when(c + _LOOK < _NCHUNK)
            def _():
                wait_idx(c + _LOOK, (r + _LOOK) % _DEPTH)
                start_rows((r + _LOOK) % _DEPTH)

            @pl.when(c < _NCHUNK)
            def _():
                wait_rows(r)

                @pl.when(c >= _DEPTH)
                def _():
                    pltpu.make_async_copy(outs[r], out_hbm.at[sl],
                                          sem_out[r]).wait()

                compute(r)
                pltpu.async_copy(outs[r], out_hbm.at[sl], sem_out[r])

            @pl.when(c + _DEPTH < _NCHUNK)
            def _():
                start_idx(c + _DEPTH, r)
        return carry

    lax.fori_loop(0, (_NCHUNK + _DEPTH - 1) // _DEPTH, ring_body, 0)
    for r0 in range(_DEPTH):
        c_last = _NCHUNK - _DEPTH + r0
        pltpu.make_async_copy(
            outs[c_last % _DEPTH],
            out_hbm.at[pl.ds(base + c_last * _C, _C)],
            sem_out[c_last % _DEPTH]).wait()


@jax.jit
def _sc_edge_score(node_emb, relp, src, dst, rel_idx):
    mesh = plsc.VectorSubcoreMesh(core_axis_name="c", subcore_axis_name="s")

    def body(node_hbm, relp_hbm, src_hbm, dst_hbm, ridx_hbm, out_hbm,
             rel_res,
             s0, s1, s2, s3, d0, d1, d2, d3, ri0, ri1, ri2, ri3,
             o0, o1, o2, o3, h0, h1, h2, h3, t0, t1, t2, t3,
             mr0, mr1, mr2, mr3, mi0, mi1, mi2, mi3, mo0, mo1, mo2, mo3):
        _edge_score_body(node_hbm, relp_hbm, src_hbm, dst_hbm, ridx_hbm,
                         out_hbm, rel_res,
                         (s0, s1, s2, s3), (d0, d1, d2, d3),
                         (ri0, ri1, ri2, ri3), (o0, o1, o2, o3),
                         (h0, h1, h2, h3), (t0, t1, t2, t3),
                         (mr0, mr1, mr2, mr3), (mi0, mi1, mi2, mi3),
                         (mo0, mo1, mo2, mo3))

    run = pl.kernel(
        body,
        mesh=mesh,
        compiler_params=pltpu.CompilerParams(needs_layout_passes=False),
        out_type=jax.ShapeDtypeStruct((_N_EDGES,), jnp.float32),
        scratch_types=[
            pltpu.VMEM((_N_RELS * _HID // 2,), jnp.float32),
        ] + [pltpu.VMEM((_C,), jnp.int32)] * 8
          + [pltpu.VMEM((_C + 16,), jnp.int32)] * 4
          + [pltpu.VMEM((_C,), jnp.float32)] * 4
          + [pltpu.VMEM((_C, _HID), jnp.float32)] * 8
          + [pltpu.SemaphoreType.DMA] * 12,
    )
    return run(node_emb, relp, src, dst, rel_idx)


def _pack_pairs(table):
    n, h = table.shape
    b16 = table.astype(jnp.bfloat16).reshape(n, h // 32, 2, 16)
    b16 = b16.transpose(0, 1, 3, 2)
    return lax.bitcast_convert_type(b16, jnp.float32).reshape(n, h // 2)


def kernel(node_emb, rel_emb, src, dst, rel_idx):
    node_packed = _pack_pairs(node_emb)
    node_packed = jnp.concatenate(
        [node_packed, jnp.zeros_like(node_packed)], axis=1)
    relp = _pack_pairs(rel_emb).reshape(-1)
    return _sc_edge_score(node_packed, relp,
                          src.astype(jnp.int32), dst.astype(jnp.int32),
                          rel_idx.astype(jnp.int32))


# speedup vs baseline: 1.4811x; 1.4811x over previous
import functools

import jax
import jax.numpy as jnp
from jax import lax
from jax.experimental import pallas as pl
from jax.experimental.pallas import tpu as pltpu
from jax.experimental.pallas import tpu_sc as plsc

_N_EDGES = 320000
_N_RELS = 1000
_HID = 128
_D2 = _HID // 2
_NW = 32
_EPW = _N_EDGES // _NW
_C = 40
_NCHUNK = _EPW // _C
_DEPTH = 4


def _edge_score_body(node_hbm, relp_hbm, src_hbm, dst_hbm, ridx_hbm, out_hbm,
                     rel_res, cidx, rids, outs, rows,
                     sem_rows, sem_idx, sem_out):
    cid = lax.axis_index("c")
    sid = lax.axis_index("s")
    wid = sid * 2 + cid
    base = wid * _EPW

    pltpu.sync_copy(relp_hbm, rel_res)

    def start_idx(c, r):
        sl = pl.ds(base + c * _C, _C)
        pltpu.async_copy(src_hbm.at[sl], cidx[r].at[pl.ds(0, _C)],
                         sem_idx[r])
        pltpu.async_copy(dst_hbm.at[sl], cidx[r].at[pl.ds(_C, _C)],
                         sem_idx[r])
        pltpu.async_copy(ridx_hbm.at[sl], rids[r].at[pl.ds(0, _C)],
                         sem_idx[r])

    def wait_idx(c, r):
        sl = pl.ds(base + c * _C, _C)
        pltpu.make_async_copy(src_hbm.at[sl], cidx[r].at[pl.ds(0, _C)],
                              sem_idx[r]).wait()
        pltpu.make_async_copy(dst_hbm.at[sl], cidx[r].at[pl.ds(_C, _C)],
                              sem_idx[r]).wait()
        pltpu.make_async_copy(ridx_hbm.at[sl], rids[r].at[pl.ds(0, _C)],
                              sem_idx[r]).wait()

    def start_rows(r):
        pltpu.async_copy(node_hbm.at[cidx[r]], rows[r], sem_rows[r])

    def wait_rows(r):
        pltpu.make_async_copy(node_hbm.at[cidx[r]], rows[r],
                              sem_rows[r]).wait()

    lane = lax.iota(jnp.int32, 16)
    low4 = lane < 4

    def compute(r):
        head_v = tail_v = rows[r]
        rid_v, out_v = rids[r], outs[r]

        def block_body(b, carry2):
            ids = rid_v[pl.ds(b * 4, 16)]
            vb = jnp.zeros((16,), jnp.float32)
            for u in range(4):
                k = b * 4 + u
                rid = jnp.take_along_axis(
                    ids, jnp.full((16,), u, jnp.int32), axis=0)
                rbase = rid * (_HID // 2) + lane
                bc = lambda v: plsc.bitcast(v, jnp.bfloat16)
                rp = [bc(plsc.load_gather(rel_res, [rbase + q * 16]))
                      for q in range(4)]
                hp = [bc(head_v[k, pl.ds(q * 16, 16)]) for q in range(4)]
                tp = [bc(tail_v[_C + k, pl.ds(q * 16, 16)])
                      for q in range(4)]
                terms = []
                for q in range(2):
                    fwd = (hp[q] * rp[q]) * tp[q + 2]
                    bwd = (tp[q] * rp[q + 2]) * hp[q + 2]
                    terms.extend(plsc.unpack(
                        fwd, format=plsc.PackFormat.INTERLEAVED))
                    terms.extend(plsc.unpack(
                        bwd, format=plsc.PackFormat.INTERLEAVED))
                while len(terms) > 1:
                    terms = [a + b2 for a, b2 in
                             zip(terms[::2], terms[1::2])]
                s = jnp.full((16,), jnp.sum(terms[0]))
                vb = jnp.where(lane == u, s, vb)
            score = jnp.clip(0.5 * vb, -20.0, 20.0)
            plsc.store_scatter(out_v, [b * 4 + lane], score, mask=low4)
            return carry2

        lax.fori_loop(0, _C // 4, block_body, 0)

    _LOOK = 2
    for r0 in range(_DEPTH):
        start_idx(r0, r0)
    for r0 in range(_LOOK):
        wait_idx(r0, r0)
        start_rows(r0)

    def ring_body(i, carry):
        for r in range(_DEPTH):
            c = _DEPTH * i + r
            sl = pl.ds(base + c * _C, _C)

            @pl.when(c + _LOOK < _NCHUNK)
            def _():
                wait_idx(c + _LOOK, (r + _LOOK) % _DEPTH)
                start_rows((r + _LOOK) % _DEPTH)

            @pl.when(c < _NCHUNK)
            def _():
                wait_rows(r)

                @pl.when(c >= _DEPTH)
                def _():
                    pltpu.make_async_copy(outs[r], out_hbm.at[sl],
                                          sem_out[r]).wait()

                compute(r)
                pltpu.async_copy(outs[r], out_hbm.at[sl], sem_out[r])

            @pl.when(c + _DEPTH < _NCHUNK)
            def _():
                start_idx(c + _DEPTH, r)
        return carry

    lax.fori_loop(0, (_NCHUNK + _DEPTH - 1) // _DEPTH, ring_body, 0)
    for r0 in range(_DEPTH):
        c_last = _NCHUNK - _DEPTH + r0
        pltpu.make_async_copy(
            outs[c_last % _DEPTH],
            out_hbm.at[pl.ds(base + c_last * _C, _C)],
            sem_out[c_last % _DEPTH]).wait()


@jax.jit
def _sc_edge_score(node_emb, relp, src, dst, rel_idx):
    mesh = plsc.VectorSubcoreMesh(core_axis_name="c", subcore_axis_name="s")

    def body(node_hbm, relp_hbm, src_hbm, dst_hbm, ridx_hbm, out_hbm,
             rel_res,
             x0, x1, x2, x3, ri0, ri1, ri2, ri3,
             o0, o1, o2, o3, w0, w1, w2, w3,
             mr0, mr1, mr2, mr3, mi0, mi1, mi2, mi3, mo0, mo1, mo2, mo3):
        _edge_score_body(node_hbm, relp_hbm, src_hbm, dst_hbm, ridx_hbm,
                         out_hbm, rel_res,
                         (x0, x1, x2, x3),
                         (ri0, ri1, ri2, ri3), (o0, o1, o2, o3),
                         (w0, w1, w2, w3),
                         (mr0, mr1, mr2, mr3), (mi0, mi1, mi2, mi3),
                         (mo0, mo1, mo2, mo3))

    run = pl.kernel(
        body,
        mesh=mesh,
        compiler_params=pltpu.CompilerParams(needs_layout_passes=False),
        out_type=jax.ShapeDtypeStruct((_N_EDGES,), jnp.float32),
        scratch_types=[
            pltpu.VMEM((_N_RELS * _HID // 2,), jnp.float32),
        ] + [pltpu.VMEM((2 * _C,), jnp.int32)] * 4
          + [pltpu.VMEM((_C + 16,), jnp.int32)] * 4
          + [pltpu.VMEM((_C,), jnp.float32)] * 4
          + [pltpu.VMEM((2 * _C, _HID), jnp.float32)] * 4
          + [pltpu.SemaphoreType.DMA] * 12,
    )
    return run(node_emb, relp, src, dst, rel_idx)


def _pack_pairs(table):
    n, h = table.shape
    b16 = table.astype(jnp.bfloat16).reshape(n, h // 32, 2, 16)
    b16 = b16.transpose(0, 1, 3, 2)
    return lax.bitcast_convert_type(b16, jnp.float32).reshape(n, h // 2)


def kernel(node_emb, rel_emb, src, dst, rel_idx):
    node_packed = _pack_pairs(node_emb)
    node_packed = jnp.concatenate(
        [node_packed, jnp.zeros_like(node_packed)], axis=1)
    relp = _pack_pairs(rel_emb).reshape(-1)
    return _sc_edge_score(node_packed, relp,
                          src.astype(jnp.int32), dst.astype(jnp.int32),
                          rel_idx.astype(jnp.int32))


# speedup vs baseline: 1.6946x; 1.1441x over previous
import jax
import jax.numpy as jnp
from jax import lax
from jax.experimental import pallas as pl
from jax.experimental.pallas import tpu as pltpu
from jax.experimental.pallas import tpu_sc as plsc

_N_EDGES = 320000
_N_RELS = 1000
_HID = 128
_D2 = _HID // 2
_NW = 32
_EPW = _N_EDGES // _NW
_C = 40
_NCHUNK = _EPW // _C
_DEPTH = 4


def _edge_score_body(node_hbm, relp_hbm, src_hbm, dst_hbm, ridx_hbm, out_hbm,
                     rel_res, cidx, rids, outs, rows,
                     sem_rows, sem_idx, sem_out):
    cid = lax.axis_index("c")
    sid = lax.axis_index("s")
    wid = sid * 2 + cid
    base = wid * _EPW

    pltpu.sync_copy(relp_hbm, rel_res)

    def start_idx(c, r):
        sl = pl.ds(base + c * _C, _C)
        pltpu.async_copy(src_hbm.at[sl], cidx[r].at[pl.ds(0, _C)],
                         sem_idx[r])
        pltpu.async_copy(dst_hbm.at[sl], cidx[r].at[pl.ds(_C, _C)],
                         sem_idx[r])
        pltpu.async_copy(ridx_hbm.at[sl], rids[r].at[pl.ds(0, _C)],
                         sem_idx[r])

    def wait_idx(c, r):
        sl = pl.ds(base + c * _C, _C)
        pltpu.make_async_copy(src_hbm.at[sl], cidx[r].at[pl.ds(0, _C)],
                              sem_idx[r]).wait()
        pltpu.make_async_copy(dst_hbm.at[sl], cidx[r].at[pl.ds(_C, _C)],
                              sem_idx[r]).wait()
        pltpu.make_async_copy(ridx_hbm.at[sl], rids[r].at[pl.ds(0, _C)],
                              sem_idx[r]).wait()

    def start_rows(r):
        pltpu.async_copy(node_hbm.at[cidx[r]], rows[r], sem_rows[r])

    def wait_rows(r):
        pltpu.make_async_copy(node_hbm.at[cidx[r]], rows[r],
                              sem_rows[r]).wait()

    lane = lax.iota(jnp.int32, 16)
    low4 = lane < 4

    def compute(r):
        head_v = tail_v = rows[r]
        rid_v, out_v = rids[r], outs[r]

        def block_body(b, carry2):
            ids = rid_v[pl.ds(b * 4, 16)]
            vb = jnp.zeros((16,), jnp.float32)
            for u in range(4):
                k = b * 4 + u
                rid = jnp.take_along_axis(
                    ids, jnp.full((16,), u, jnp.int32), axis=0)
                rbase = rid * (_HID // 2) + lane
                bc = lambda v: plsc.bitcast(v, jnp.bfloat16)
                rp = [bc(plsc.load_gather(rel_res, [rbase + q * 16]))
                      for q in range(4)]
                hp = [bc(head_v[k, pl.ds(q * 16, 16)]) for q in range(4)]
                tp = [bc(tail_v[_C + k, pl.ds(q * 16, 16)])
                      for q in range(4)]
                terms = []
                for q in range(2):
                    fwd = (hp[q] * rp[q]) * tp[q + 2]
                    bwd = (tp[q] * rp[q + 2]) * hp[q + 2]
                    terms.extend(plsc.unpack(
                        fwd, format=plsc.PackFormat.INTERLEAVED))
                    terms.extend(plsc.unpack(
                        bwd, format=plsc.PackFormat.INTERLEAVED))
                while len(terms) > 1:
                    terms = [a + b2 for a, b2 in
                             zip(terms[::2], terms[1::2])]
                s = jnp.full((16,), jnp.sum(terms[0]))
                vb = jnp.where(lane == u, s, vb)
            score = jnp.clip(0.5 * vb, -20.0, 20.0)
            plsc.store_scatter(out_v, [b * 4 + lane], score, mask=low4)
            return carry2

        lax.fori_loop(0, _C // 4, block_body, 0)

    _LOOK = 2
    for r0 in range(_DEPTH):
        start_idx(r0, r0)
    for r0 in range(_LOOK):
        wait_idx(r0, r0)
        start_rows(r0)

    def ring_body(i, carry):
        for r in range(_DEPTH):
            c = _DEPTH * i + r
            sl = pl.ds(base + c * _C, _C)

            @pl.when(c + _LOOK < _NCHUNK)
            def _():
                wait_idx(c + _LOOK, (r + _LOOK) % _DEPTH)
                start_rows((r + _LOOK) % _DEPTH)

            @pl.when(c < _NCHUNK)
            def _():
                wait_rows(r)

                @pl.when(c >= _DEPTH)
                def _():
                    pltpu.make_async_copy(outs[r], out_hbm.at[sl],
                                          sem_out[r]).wait()

                compute(r)
                pltpu.async_copy(outs[r], out_hbm.at[sl], sem_out[r])

            @pl.when(c + _DEPTH < _NCHUNK)
            def _():
                start_idx(c + _DEPTH, r)
        return carry

    lax.fori_loop(0, (_NCHUNK + _DEPTH - 1) // _DEPTH, ring_body, 0)
    for r0 in range(_DEPTH):
        c_last = _NCHUNK - _DEPTH + r0
        pltpu.make_async_copy(
            outs[c_last % _DEPTH],
            out_hbm.at[pl.ds(base + c_last * _C, _C)],
            sem_out[c_last % _DEPTH]).wait()


@jax.jit
def _sc_edge_score(node_emb, relp, src, dst, rel_idx):
    mesh = plsc.VectorSubcoreMesh(core_axis_name="c", subcore_axis_name="s")

    def body(node_hbm, relp_hbm, src_hbm, dst_hbm, ridx_hbm, out_hbm,
             rel_res,
             x0, x1, x2, x3, ri0, ri1, ri2, ri3,
             o0, o1, o2, o3, w0, w1, w2, w3,
             mr0, mr1, mr2, mr3, mi0, mi1, mi2, mi3, mo0, mo1, mo2, mo3):
        _edge_score_body(node_hbm, relp_hbm, src_hbm, dst_hbm, ridx_hbm,
                         out_hbm, rel_res,
                         (x0, x1, x2, x3),
                         (ri0, ri1, ri2, ri3), (o0, o1, o2, o3),
                         (w0, w1, w2, w3),
                         (mr0, mr1, mr2, mr3), (mi0, mi1, mi2, mi3),
                         (mo0, mo1, mo2, mo3))

    run = pl.kernel(
        body,
        mesh=mesh,
        compiler_params=pltpu.CompilerParams(
            needs_layout_passes=False, use_tc_tiling_on_sc=False),
        out_type=jax.ShapeDtypeStruct((_N_EDGES,), jnp.float32),
        scratch_types=[
            pltpu.VMEM((_N_RELS * _HID // 2,), jnp.float32),
        ] + [pltpu.VMEM((2 * _C,), jnp.int32)] * 4
          + [pltpu.VMEM((_C + 16,), jnp.int32)] * 4
          + [pltpu.VMEM((_C,), jnp.float32)] * 4
          + [pltpu.VMEM((2 * _C, _HID // 2), jnp.float32)] * 4
          + [pltpu.SemaphoreType.DMA] * 12,
    )
    return run(node_emb, relp, src, dst, rel_idx)


def _pack_pairs(table):
    n, h = table.shape
    b16 = table.astype(jnp.bfloat16).reshape(n, h // 32, 2, 16)
    b16 = b16.transpose(0, 1, 3, 2)
    return lax.bitcast_convert_type(b16, jnp.float32).reshape(n, h // 2)


def kernel(node_emb, rel_emb, src, dst, rel_idx):
    node_packed = _pack_pairs(node_emb)
    relp = _pack_pairs(rel_emb).reshape(-1)
    return _sc_edge_score(node_packed, relp,
                          src.astype(jnp.int32), dst.astype(jnp.int32),
                          rel_idx.astype(jnp.int32))
